# baseline (speedup 1.0000x reference)
import functools

import jax
import jax.numpy as jnp
from jax import lax
from jax.experimental import pallas as pl
from jax.experimental.pallas import tpu as pltpu, tpu_sc as plsc


def _gather_kernel(B_total, D, n_workers, num_cores, chunk):
    b_per_w = B_total // n_workers
    n_chunks = b_per_w // chunk
    assert n_chunks % 2 == 0 and n_chunks >= 4
    mesh = plsc.VectorSubcoreMesh(core_axis_name="c", subcore_axis_name="s")

    @functools.partial(
        pl.kernel,
        mesh=mesh,
        out_type=jax.ShapeDtypeStruct((B_total, D), jnp.float32),
        scratch_types=[
            pltpu.VMEM((chunk,), jnp.int32),
            pltpu.VMEM((chunk,), jnp.int32),
            pltpu.VMEM((chunk, D), jnp.float32),
            pltpu.VMEM((chunk, D), jnp.float32),
            pltpu.SemaphoreType.DMA,
            pltpu.SemaphoreType.DMA,
            pltpu.SemaphoreType.DMA,
            pltpu.SemaphoreType.DMA,
        ],
        compiler_params=pltpu.CompilerParams(use_tc_tiling_on_sc=False),
    )
    def k(table_hbm, idx_hbm, out_hbm,
          idx_v0, idx_v1, rows_v0, rows_v1,
          sem_g0, sem_g1, sem_w0, sem_w1):
        idx_v = (idx_v0, idx_v1)
        rows_v = (rows_v0, rows_v1)
        sem_g = (sem_g0, sem_g1)
        sem_w = (sem_w0, sem_w1)

        wid = lax.axis_index("s") * num_cores + lax.axis_index("c")
        base = wid * b_per_w

        def gather_copy(b):
            return pltpu.make_async_copy(
                table_hbm.at[idx_v[b]], rows_v[b], sem_g[b])

        def write_copy(b, off):
            return pltpu.make_async_copy(
                rows_v[b], out_hbm.at[pl.ds(off, chunk)], sem_w[b])

        for b in range(2):
            pltpu.sync_copy(idx_hbm.at[pl.ds(base + b * chunk, chunk)],
                            idx_v[b])
            gather_copy(b).start()

        def body(t, carry):
            for b in range(2):
                g = 2 * t + b
                off = base + g * chunk
                gather_copy(b).wait()
                write_copy(b, off).start()
                pltpu.sync_copy(
                    idx_hbm.at[pl.ds(off + 2 * chunk, chunk)], idx_v[b])
                write_copy(b, off).wait()
                gather_copy(b).start()
            return carry

        lax.fori_loop(0, n_chunks // 2 - 1, body, 0)

        for b in range(2):
            g = n_chunks - 2 + b
            off = base + g * chunk
            gather_copy(b).wait()
            write_copy(b, off).start()
        for b in range(2):
            off = base + (n_chunks - 2 + b) * chunk
            write_copy(b, off).wait()

    return k


def kernel(overlap, scene, embed_table):
    B, O = overlap.shape
    V, D = embed_table.shape
    B_total = B * O
    idx_flat = overlap.reshape(B_total).astype(jnp.int32)

    info = plsc.get_sparse_core_info()
    n_workers = info.num_cores * info.num_subcores
    chunk = 800

    k = _gather_kernel(B_total, D, n_workers, info.num_cores, chunk)
    out = k(embed_table, idx_flat)
    return out.reshape(B, O, D)
